# baseline (speedup 1.0000x reference)
import functools

import jax
import jax.numpy as jnp
from jax import lax
from jax.experimental import pallas as pl
from jax.experimental.pallas import tpu as pltpu
from jax.experimental.pallas import tpu_sc as plsc

D_MODEL = 512


@functools.lru_cache(maxsize=None)
def _build_lookup(total, d_model, vocab):
    info = plsc.get_sparse_core_info()
    num_cores, num_subcores = info.num_cores, info.num_subcores
    nw = num_cores * num_subcores
    b_per_w = total // nw

    mesh = plsc.VectorSubcoreMesh(core_axis_name="c", subcore_axis_name="s")

    @functools.partial(
        pl.kernel,
        mesh=mesh,
        out_type=jax.ShapeDtypeStruct((total * d_model,), jnp.float32),
        scratch_types=[
            pltpu.SemaphoreType.DMA,
        ],
    )
    def lookup(idx_hbm, table_hbm, out_hbm, dsem):
        wid = lax.axis_index("s") * num_cores + lax.axis_index("c")
        base = wid * b_per_w

        def body(j, carry):
            row = base + j
            src = (row & 16383) * d_model
            pltpu.make_async_copy(
                table_hbm.at[pl.ds(src, d_model)],
                out_hbm.at[pl.ds(row * d_model, d_model)],
                dsem,
            ).start()
            return carry

        lax.fori_loop(0, b_per_w, body, 0)
        pltpu.make_async_copy(
            table_hbm.at[pl.ds(0, b_per_w * d_model)],
            out_hbm.at[pl.ds(base * d_model, b_per_w * d_model)],
            dsem,
        ).wait()

    return lookup, nw, b_per_w


def kernel(x, weight):
    batch, hist = x.shape
    total = batch * hist
    vocab, d_model = weight.shape
    lookup, nw, b_per_w = _build_lookup(total, d_model, vocab)
    idx = x.reshape(nw, b_per_w).astype(jnp.int32)
    out = lookup(idx, weight.reshape(-1))
    return out.reshape(batch, hist, d_model)


# speedup vs baseline: 18.0709x; 18.0709x over previous
import functools

import jax
import jax.numpy as jnp
from jax import lax
from jax.experimental import pallas as pl
from jax.experimental.pallas import tpu as pltpu
from jax.experimental.pallas import tpu_sc as plsc

D_MODEL = 512
CHUNK = 40
NBUF = 5


@functools.lru_cache(maxsize=None)
def _build_lookup(total, d_model):
    info = plsc.get_sparse_core_info()
    num_cores, num_subcores = info.num_cores, info.num_subcores
    nw = num_cores * num_subcores
    assert total % (nw * CHUNK) == 0
    b_per_w = total // nw
    n_chunks = b_per_w // CHUNK
    assert n_chunks % NBUF == 0
    n_groups = n_chunks // NBUF

    mesh = plsc.VectorSubcoreMesh(core_axis_name="c", subcore_axis_name="s")

    @functools.partial(
        pl.kernel,
        mesh=mesh,
        compiler_params=pltpu.CompilerParams(use_tc_tiling_on_sc=False),
        out_type=jax.ShapeDtypeStruct((total, d_model), jnp.float32),
        scratch_types=[
            pltpu.VMEM((b_per_w,), jnp.int32),
            pltpu.VMEM((NBUF, CHUNK, d_model), jnp.float32),
        ]
        + [pltpu.SemaphoreType.DMA] * (2 * NBUF),
    )
    def lookup(idx_hbm, table_hbm, out_hbm, idx_v, rows_v, *sems):
        gsem, wsem = sems[:NBUF], sems[NBUF:]
        wid = lax.axis_index("s") * num_cores + lax.axis_index("c")
        base = wid * b_per_w
        pltpu.sync_copy(idx_hbm.at[wid], idx_v)

        def gather(c, b):
            return pltpu.make_async_copy(
                table_hbm.at[idx_v.at[pl.ds(c * CHUNK, CHUNK)]],
                rows_v.at[b],
                gsem[b],
            )

        def write(c, b):
            return pltpu.make_async_copy(
                rows_v.at[b], out_hbm.at[pl.ds(base + c * CHUNK, CHUNK)], wsem[b]
            )

        for b in range(NBUF):
            gather(b, b).start()

        def body(i, carry):
            c0 = NBUF * i
            for b in range(NBUF):
                gather(c0 + b, b).wait()
                write(c0 + b, b).start()

            @pl.when(i + 1 < n_groups)
            def _():
                for b in range(NBUF):
                    write(c0 + b, b).wait()
                    gather(c0 + NBUF + b, b).start()

            return carry

        lax.fori_loop(0, n_groups, body, 0)
        for b in range(NBUF):
            write(n_chunks - NBUF + b, b).wait()

    return lookup, nw, b_per_w


def kernel(x, weight):
    batch, hist = x.shape
    total = batch * hist
    d_model = weight.shape[1]
    lookup, nw, b_per_w = _build_lookup(total, d_model)
    idx = x.reshape(nw, b_per_w).astype(jnp.int32)
    out = lookup(idx, weight)
    return out.reshape(batch, hist, d_model)


# speedup vs baseline: 43.6241x; 2.4141x over previous
import functools

import jax
import jax.numpy as jnp
from jax import lax
from jax.experimental import pallas as pl
from jax.experimental.pallas import tpu as pltpu
from jax.experimental.pallas import tpu_sc as plsc

D_MODEL = 512
CHUNK = 40
NBUF = 5


@functools.lru_cache(maxsize=None)
def _build_lookup(total, d_model):
    info = plsc.get_sparse_core_info()
    num_cores, num_subcores = info.num_cores, info.num_subcores
    nw = num_cores * num_subcores
    assert total % (nw * CHUNK) == 0
    b_per_w = total // nw
    n_chunks = b_per_w // CHUNK
    assert n_chunks % NBUF == 0
    n_groups = n_chunks // NBUF

    mesh = plsc.VectorSubcoreMesh(core_axis_name="c", subcore_axis_name="s")

    @functools.partial(
        pl.kernel,
        mesh=mesh,
        out_type=jax.ShapeDtypeStruct((total, d_model), jnp.float32),
        scratch_types=[
            pltpu.VMEM((b_per_w,), jnp.int32),
            pltpu.VMEM((NBUF, CHUNK, d_model), jnp.float32),
        ]
        + [pltpu.SemaphoreType.DMA] * (2 * NBUF),
    )
    def lookup(idx_hbm, table_hbm, out_hbm, idx_v, rows_v, *sems):
        gsem, wsem = sems[:NBUF], sems[NBUF:]
        wid = lax.axis_index("s") * num_cores + lax.axis_index("c")
        base = wid * b_per_w
        pltpu.sync_copy(idx_hbm.at[wid], idx_v)

        def gather(c, b):
            return pltpu.make_async_copy(
                table_hbm.at[idx_v.at[pl.ds(c * CHUNK, CHUNK)]],
                rows_v.at[b],
                gsem[b],
            )

        def write(c, b):
            return pltpu.make_async_copy(
                rows_v.at[b], out_hbm.at[pl.ds(base + c * CHUNK, CHUNK)], wsem[b]
            )

        for b in range(NBUF):
            gather(b, b).start()

        def body(i, carry):
            c0 = NBUF * i
            for b in range(NBUF):
                gather(c0 + b, b).wait()
                write(c0 + b, b).start()

            @pl.when(i + 1 < n_groups)
            def _():
                for b in range(NBUF):
                    write(c0 + b, b).wait()
                    gather(c0 + NBUF + b, b).start()

            return carry

        lax.fori_loop(0, n_groups, body, 0)
        for b in range(NBUF):
            write(n_chunks - NBUF + b, b).wait()

    return lookup, nw, b_per_w


def kernel(x, weight):
    batch, hist = x.shape
    total = batch * hist
    d_model = weight.shape[1]
    lookup, nw, b_per_w = _build_lookup(total, d_model)
    idx = x.reshape(nw, b_per_w).astype(jnp.int32)
    out = lookup(idx, weight)
    return out.reshape(batch, hist, d_model)
